# baseline (speedup 1.0000x reference)
import functools

import jax
import jax.numpy as jnp
from jax import lax
from jax.experimental import pallas as pl
from jax.experimental.pallas import tpu as pltpu
from jax.experimental.pallas import tpu_sc as plsc

_N = 10000
_E = 320000
_G = 64
_NP = 10240
_DH = 256
_HF = 128

_BR = 512

_NSUB = 16
_K = 80
_EPT = _E // _NSUB
_RPT = _NP // _NSUB
_DW = 32

_HIGH = jax.lax.Precision.HIGHEST


def _sc_deg(dst):
    mesh = plsc.VectorSubcoreMesh(core_axis_name="c", subcore_axis_name="s")
    zer = jnp.zeros((_RPT, _HF), jnp.float32)
    half = _E // 2
    ept = half // _NSUB
    nit = ept // _K

    @functools.partial(
        pl.kernel,
        mesh=mesh,
        out_type=[
            jax.ShapeDtypeStruct((_NP, _HF), jnp.float32),
            jax.ShapeDtypeStruct((_NP, _HF), jnp.float32),
        ],
        scratch_types=[
            pltpu.VMEM((_K,), jnp.int32),
            pltpu.VMEM((_K, _HF), jnp.float32),
            pltpu.VMEM_SHARED((_NP, _HF), jnp.float32),
        ],
    )
    def k(dst_hbm, zer_hbm, out0, out1, didx, ones_v, acc):
        c = lax.axis_index("c")
        s = lax.axis_index("s")
        r0 = s * _RPT
        pltpu.sync_copy(zer_hbm, acc.at[pl.ds(r0, _RPT)])

        def fill(j, carry):
            for t in range(_HF // 16):
                ones_v[j, pl.ds(t * 16, 16)] = jnp.ones((16,), jnp.float32)
            return carry

        lax.fori_loop(0, _K, fill, 0)
        plsc.subcore_barrier()

        base0 = c * half + s * ept

        def body(i, carry):
            base = base0 + i * _K
            pltpu.sync_copy(dst_hbm.at[pl.ds(base, _K)], didx)
            pltpu.sync_copy(ones_v, acc.at[didx], add=True)
            return carry

        lax.fori_loop(0, nit, body, 0)
        plsc.subcore_barrier()

        @pl.when(c == 0)
        def _():
            pltpu.sync_copy(acc.at[pl.ds(r0, _RPT)], out0.at[pl.ds(r0, _RPT)])

        @pl.when(c == 1)
        def _():
            pltpu.sync_copy(acc.at[pl.ds(r0, _RPT)], out1.at[pl.ds(r0, _RPT)])

    return k(dst, zer)


def _sc_agg(glo, ghi, src, dst):
    mesh = plsc.VectorSubcoreMesh(core_axis_name="c", subcore_axis_name="s")
    zer = jnp.zeros((_RPT, _HF), jnp.float32)
    nit = _EPT // _K

    @functools.partial(
        pl.kernel,
        mesh=mesh,
        out_type=[
            jax.ShapeDtypeStruct((_NP, _HF), jnp.float32),
            jax.ShapeDtypeStruct((_NP, _HF), jnp.float32),
        ],
        scratch_types=[
            pltpu.VMEM((_K,), jnp.int32),
            pltpu.VMEM((_K,), jnp.int32),
            pltpu.VMEM((_K, _HF), jnp.float32),
            pltpu.VMEM_SHARED((_NP, _HF), jnp.float32),
            pltpu.SemaphoreType.DMA,
        ],
    )
    def k(glo_hbm, ghi_hbm, src_hbm, dst_hbm, zer_hbm, out0, out1,
          sidx, didx, rows, acc, sem):
        c = lax.axis_index("c")
        s = lax.axis_index("s")
        r0 = s * _RPT
        pltpu.sync_copy(zer_hbm, acc.at[pl.ds(r0, _RPT)])
        plsc.subcore_barrier()

        def run(g_hbm):
            def body(i, carry):
                base = s * _EPT + i * _K
                pltpu.sync_copy(src_hbm.at[pl.ds(base, _K)], sidx)
                pltpu.sync_copy(dst_hbm.at[pl.ds(base, _K)], didx)
                pltpu.async_copy(g_hbm.at[sidx], rows, sem).wait()
                pltpu.sync_copy(rows, acc.at[didx], add=True)
                return carry

            lax.fori_loop(0, nit, body, 0)

        @pl.when(c == 0)
        def _():
            run(glo_hbm)

        @pl.when(c == 1)
        def _():
            run(ghi_hbm)

        plsc.subcore_barrier()

        @pl.when(c == 0)
        def _():
            pltpu.sync_copy(acc.at[pl.ds(r0, _RPT)], out0.at[pl.ds(r0, _RPT)])

        @pl.when(c == 1)
        def _():
            pltpu.sync_copy(acc.at[pl.ds(r0, _RPT)], out1.at[pl.ds(r0, _RPT)])

    return k(glo, ghi, src, dst, zer)


def _prep1_body(x_ref, w_ref, d0_ref, d1_ref, dinv_ref, glo_ref, ghi_ref):
    i = pl.program_id(0)
    deg = d0_ref[...] + d1_ref[...] + 1.0
    rows = i * _BR + lax.broadcasted_iota(jnp.int32, (_BR, 1), 0)
    dinv = jnp.where(rows < _N, lax.rsqrt(deg), 0.0)
    dinv_ref[...] = dinv
    g = jnp.dot(x_ref[...], w_ref[...],
                preferred_element_type=jnp.float32, precision=_HIGH) * dinv
    glo_ref[...] = g[:, :_HF]
    ghi_ref[...] = g[:, _HF:]


def _prep1(x_pad, w1, d0, d1):
    return pl.pallas_call(
        _prep1_body,
        grid=(_NP // _BR,),
        in_specs=[
            pl.BlockSpec((_BR, 128), lambda i: (i, 0)),
            pl.BlockSpec((128, _DH), lambda i: (0, 0)),
            pl.BlockSpec((_BR, 1), lambda i: (i, 0)),
            pl.BlockSpec((_BR, 1), lambda i: (i, 0)),
        ],
        out_specs=[
            pl.BlockSpec((_BR, 1), lambda i: (i, 0)),
            pl.BlockSpec((_BR, _HF), lambda i: (i, 0)),
            pl.BlockSpec((_BR, _HF), lambda i: (i, 0)),
        ],
        out_shape=[
            jax.ShapeDtypeStruct((_NP, 1), jnp.float32),
            jax.ShapeDtypeStruct((_NP, _HF), jnp.float32),
            jax.ShapeDtypeStruct((_NP, _HF), jnp.float32),
        ],
    )(x_pad, w1, d0, d1)


def _fuse_body(s0_ref, s1_ref, gl_ref, gh_ref, dinv_ref, b_ref, w_ref,
               olo_ref, ohi_ref):
    dinv = dinv_ref[...]
    h = jnp.concatenate(
        [s0_ref[...] + gl_ref[...], s1_ref[...] + gh_ref[...]], axis=1)
    h = jnp.maximum(h * dinv + b_ref[...], 0.0)
    g = jnp.dot(h, w_ref[...],
                preferred_element_type=jnp.float32, precision=_HIGH) * dinv
    olo_ref[...] = g[:, :_HF]
    ohi_ref[...] = g[:, _HF:]


def _fuse(s0, s1, gl, gh, dinv, b, w):
    return pl.pallas_call(
        _fuse_body,
        grid=(_NP // _BR,),
        in_specs=[
            pl.BlockSpec((_BR, _HF), lambda i: (i, 0)),
            pl.BlockSpec((_BR, _HF), lambda i: (i, 0)),
            pl.BlockSpec((_BR, _HF), lambda i: (i, 0)),
            pl.BlockSpec((_BR, _HF), lambda i: (i, 0)),
            pl.BlockSpec((_BR, 1), lambda i: (i, 0)),
            pl.BlockSpec((1, _DH), lambda i: (0, 0)),
            pl.BlockSpec((_DH, _DH), lambda i: (0, 0)),
        ],
        out_specs=[
            pl.BlockSpec((_BR, _HF), lambda i: (i, 0)),
            pl.BlockSpec((_BR, _HF), lambda i: (i, 0)),
        ],
        out_shape=[
            jax.ShapeDtypeStruct((_NP, _HF), jnp.float32),
            jax.ShapeDtypeStruct((_NP, _HF), jnp.float32),
        ],
    )(s0, s1, gl, gh, dinv, b, w)


def _pool_body(s0_ref, s1_ref, gl_ref, gh_ref, dinv_ref, b_ref, batch_ref,
               wp_ref, bp_ref, sums_ref, cnt_ref, out_ref, *, nblk):
    i = pl.program_id(0)

    @pl.when(i == 0)
    def _init():
        sums_ref[...] = jnp.zeros_like(sums_ref)
        cnt_ref[...] = jnp.zeros_like(cnt_ref)

    h = jnp.concatenate(
        [s0_ref[...] + gl_ref[...], s1_ref[...] + gh_ref[...]], axis=1)
    h = h * dinv_ref[...] + b_ref[...]
    v = jnp.dot(h, wp_ref[...],
                preferred_element_type=jnp.float32, precision=_HIGH)
    ids = lax.broadcasted_iota(jnp.int32, (1, _G), 1)
    oh = (batch_ref[...] == ids).astype(jnp.float32)
    sums_ref[...] += lax.dot_general(
        oh, v, (((0,), (0,)), ((), ())),
        preferred_element_type=jnp.float32, precision=_HIGH)
    cnt_ref[...] += jnp.sum(oh, axis=0)[:, None]

    @pl.when(i == nblk - 1)
    def _epilogue():
        out_ref[...] = (sums_ref[...] / jnp.maximum(cnt_ref[...], 1.0)
                        + bp_ref[...])


def _pool(s0, s1, gl, gh, dinv, b, batch_pad, wp, bp):
    nblk = _NP // _BR
    body = functools.partial(_pool_body, nblk=nblk)
    _, _, out = pl.pallas_call(
        body,
        grid=(nblk,),
        in_specs=[
            pl.BlockSpec((_BR, _HF), lambda i: (i, 0)),
            pl.BlockSpec((_BR, _HF), lambda i: (i, 0)),
            pl.BlockSpec((_BR, _HF), lambda i: (i, 0)),
            pl.BlockSpec((_BR, _HF), lambda i: (i, 0)),
            pl.BlockSpec((_BR, 1), lambda i: (i, 0)),
            pl.BlockSpec((1, _DH), lambda i: (0, 0)),
            pl.BlockSpec((_BR, 1), lambda i: (i, 0)),
            pl.BlockSpec((_DH, 1), lambda i: (0, 0)),
            pl.BlockSpec((1, 1), lambda i: (0, 0)),
        ],
        out_specs=[
            pl.BlockSpec((_G, 1), lambda i: (0, 0)),
            pl.BlockSpec((_G, 1), lambda i: (0, 0)),
            pl.BlockSpec((_G, 1), lambda i: (0, 0)),
        ],
        out_shape=[
            jax.ShapeDtypeStruct((_G, 1), jnp.float32),
            jax.ShapeDtypeStruct((_G, 1), jnp.float32),
            jax.ShapeDtypeStruct((_G, 1), jnp.float32),
        ],
    )(s0, s1, gl, gh, dinv, b, batch_pad, wp, bp)
    return out


def kernel(x, edge_index, batch, W1, b1, W2, b2, W3, b3, Wp, bp):
    src, dst = edge_index[0], edge_index[1]

    deg0, deg1 = _sc_deg(dst)

    x_pad = jnp.zeros((_NP, 128), jnp.float32).at[:_N].set(x)
    dinv, gl, gh = _prep1(x_pad, W1, deg0[:, :1], deg1[:, :1])

    s0, s1 = _sc_agg(gl, gh, src, dst)
    gl, gh = _fuse(s0, s1, gl, gh, dinv, b1[None, :], W2)
    s0, s1 = _sc_agg(gl, gh, src, dst)
    gl, gh = _fuse(s0, s1, gl, gh, dinv, b2[None, :], W3)
    s0, s1 = _sc_agg(gl, gh, src, dst)

    batch_pad = jnp.full((_NP, 1), _G, jnp.int32).at[:_N, 0].set(batch)
    return _pool(s0, s1, gl, gh, dinv, b3[None, :], batch_pad, Wp,
                 bp[:, None])


# speedup vs baseline: 1.6163x; 1.6163x over previous
import functools

import jax
import jax.numpy as jnp
from jax import lax
from jax.experimental import pallas as pl
from jax.experimental.pallas import tpu as pltpu
from jax.experimental.pallas import tpu_sc as plsc

_N = 10000
_E = 320000
_G = 64
_NP = 10240
_DH = 256
_HF = 128

_BR = 512

_NSUB = 16
_K = 80
_KA = 80
_EPT = _E // _NSUB
_RPT = _NP // _NSUB
_DW = 16

_HIGH = jax.lax.Precision.HIGHEST


def _sc_deg(dst):
    mesh = plsc.VectorSubcoreMesh(core_axis_name="c", subcore_axis_name="s")
    zer = jnp.zeros((_RPT, _DW), jnp.float32)
    half = _E // 2
    ept = half // _NSUB
    nit = ept // _K

    @functools.partial(
        pl.kernel,
        mesh=mesh,
        out_type=[
            jax.ShapeDtypeStruct((_NP, _DW), jnp.float32),
            jax.ShapeDtypeStruct((_NP, _DW), jnp.float32),
        ],
        scratch_types=[
            pltpu.VMEM((_K,), jnp.int32),
            pltpu.VMEM((_K, _DW), jnp.float32),
            pltpu.VMEM_SHARED((_NP, _DW), jnp.float32),
        ],
    )
    def k(dst_hbm, zer_hbm, out0, out1, didx, ones_v, acc):
        c = lax.axis_index("c")
        s = lax.axis_index("s")
        r0 = s * _RPT
        pltpu.sync_copy(zer_hbm, acc.at[pl.ds(r0, _RPT)])

        def fill(j, carry):
            ones_v[j, pl.ds(0, _DW)] = jnp.ones((_DW,), jnp.float32)
            return carry

        lax.fori_loop(0, _K, fill, 0)
        plsc.subcore_barrier()

        base0 = c * half + s * ept

        def body(i, carry):
            base = base0 + i * _K
            pltpu.sync_copy(dst_hbm.at[pl.ds(base, _K)], didx)
            pltpu.sync_copy(ones_v, acc.at[didx], add=True)
            return carry

        lax.fori_loop(0, nit, body, 0)
        plsc.subcore_barrier()

        @pl.when(c == 0)
        def _():
            pltpu.sync_copy(acc.at[pl.ds(r0, _RPT)], out0.at[pl.ds(r0, _RPT)])

        @pl.when(c == 1)
        def _():
            pltpu.sync_copy(acc.at[pl.ds(r0, _RPT)], out1.at[pl.ds(r0, _RPT)])

    return k(dst, zer)


def _sc_agg(glo, ghi, src, dst):
    mesh = plsc.VectorSubcoreMesh(core_axis_name="c", subcore_axis_name="s")
    zer = jnp.zeros((_RPT, _HF), jnp.float32)
    nit = _EPT // _KA

    @functools.partial(
        pl.kernel,
        mesh=mesh,
        out_type=[
            jax.ShapeDtypeStruct((_NP, _HF), jnp.float32),
            jax.ShapeDtypeStruct((_NP, _HF), jnp.float32),
        ],
        scratch_types=[
            pltpu.VMEM((_KA,), jnp.int32),
            pltpu.VMEM((_KA,), jnp.int32),
            pltpu.VMEM((_KA,), jnp.int32),
            pltpu.VMEM((_KA,), jnp.int32),
            pltpu.VMEM((_KA, _HF), jnp.float32),
            pltpu.VMEM((_KA, _HF), jnp.float32),
            pltpu.VMEM_SHARED((_NP, _HF), jnp.float32),
            pltpu.SemaphoreType.DMA,
            pltpu.SemaphoreType.DMA,
        ],
    )
    def k(glo_hbm, ghi_hbm, src_hbm, dst_hbm, zer_hbm, out0, out1,
          sidx0, sidx1, didx0, didx1, rows0, rows1, acc, sem0, sem1):
        c = lax.axis_index("c")
        s = lax.axis_index("s")
        r0 = s * _RPT
        pltpu.sync_copy(zer_hbm, acc.at[pl.ds(r0, _RPT)])
        plsc.subcore_barrier()

        bufs = ((sidx0, didx0, rows0, sem0), (sidx1, didx1, rows1, sem1))

        def run(g_hbm):
            for b, (sidx, didx, rows, sem) in enumerate(bufs):
                base = s * _EPT + b * _KA
                pltpu.sync_copy(src_hbm.at[pl.ds(base, _KA)], sidx)
                pltpu.sync_copy(dst_hbm.at[pl.ds(base, _KA)], didx)
                pltpu.async_copy(g_hbm.at[sidx], rows, sem)

            def body(g, carry):
                for b, (sidx, didx, rows, sem) in enumerate(bufs):
                    i = 2 * g + b
                    pltpu.make_async_copy(g_hbm.at[sidx], rows, sem).wait()
                    pltpu.sync_copy(rows, acc.at[didx], add=True)
                    nb = lax.rem(i + 2, nit)
                    base = s * _EPT + nb * _KA
                    pltpu.sync_copy(src_hbm.at[pl.ds(base, _KA)], sidx)
                    pltpu.sync_copy(dst_hbm.at[pl.ds(base, _KA)], didx)
                    pltpu.async_copy(g_hbm.at[sidx], rows, sem)
                return carry

            lax.fori_loop(0, nit // 2, body, 0)
            for sidx, didx, rows, sem in bufs:
                pltpu.make_async_copy(g_hbm.at[sidx], rows, sem).wait()

        @pl.when(c == 0)
        def _():
            run(glo_hbm)

        @pl.when(c == 1)
        def _():
            run(ghi_hbm)

        plsc.subcore_barrier()

        @pl.when(c == 0)
        def _():
            pltpu.sync_copy(acc.at[pl.ds(r0, _RPT)], out0.at[pl.ds(r0, _RPT)])

        @pl.when(c == 1)
        def _():
            pltpu.sync_copy(acc.at[pl.ds(r0, _RPT)], out1.at[pl.ds(r0, _RPT)])

    return k(glo, ghi, src, dst, zer)


def _prep1_body(x_ref, w_ref, d0_ref, d1_ref, dinv_ref, glo_ref, ghi_ref):
    i = pl.program_id(0)
    deg = d0_ref[...] + d1_ref[...] + 1.0
    rows = i * _BR + lax.broadcasted_iota(jnp.int32, (_BR, 1), 0)
    dinv = jnp.where(rows < _N, lax.rsqrt(deg), 0.0)
    dinv_ref[...] = dinv
    g = jnp.dot(x_ref[...], w_ref[...],
                preferred_element_type=jnp.float32, precision=_HIGH) * dinv
    glo_ref[...] = g[:, :_HF]
    ghi_ref[...] = g[:, _HF:]


def _prep1(x_pad, w1, d0, d1):
    return pl.pallas_call(
        _prep1_body,
        grid=(_NP // _BR,),
        in_specs=[
            pl.BlockSpec((_BR, 128), lambda i: (i, 0)),
            pl.BlockSpec((128, _DH), lambda i: (0, 0)),
            pl.BlockSpec((_BR, 1), lambda i: (i, 0)),
            pl.BlockSpec((_BR, 1), lambda i: (i, 0)),
        ],
        out_specs=[
            pl.BlockSpec((_BR, 1), lambda i: (i, 0)),
            pl.BlockSpec((_BR, _HF), lambda i: (i, 0)),
            pl.BlockSpec((_BR, _HF), lambda i: (i, 0)),
        ],
        out_shape=[
            jax.ShapeDtypeStruct((_NP, 1), jnp.float32),
            jax.ShapeDtypeStruct((_NP, _HF), jnp.float32),
            jax.ShapeDtypeStruct((_NP, _HF), jnp.float32),
        ],
    )(x_pad, w1, d0, d1)


def _fuse_body(s0_ref, s1_ref, gl_ref, gh_ref, dinv_ref, b_ref, w_ref,
               olo_ref, ohi_ref):
    dinv = dinv_ref[...]
    h = jnp.concatenate(
        [s0_ref[...] + gl_ref[...], s1_ref[...] + gh_ref[...]], axis=1)
    h = jnp.maximum(h * dinv + b_ref[...], 0.0)
    g = jnp.dot(h, w_ref[...],
                preferred_element_type=jnp.float32, precision=_HIGH) * dinv
    olo_ref[...] = g[:, :_HF]
    ohi_ref[...] = g[:, _HF:]


def _fuse(s0, s1, gl, gh, dinv, b, w):
    return pl.pallas_call(
        _fuse_body,
        grid=(_NP // _BR,),
        in_specs=[
            pl.BlockSpec((_BR, _HF), lambda i: (i, 0)),
            pl.BlockSpec((_BR, _HF), lambda i: (i, 0)),
            pl.BlockSpec((_BR, _HF), lambda i: (i, 0)),
            pl.BlockSpec((_BR, _HF), lambda i: (i, 0)),
            pl.BlockSpec((_BR, 1), lambda i: (i, 0)),
            pl.BlockSpec((1, _DH), lambda i: (0, 0)),
            pl.BlockSpec((_DH, _DH), lambda i: (0, 0)),
        ],
        out_specs=[
            pl.BlockSpec((_BR, _HF), lambda i: (i, 0)),
            pl.BlockSpec((_BR, _HF), lambda i: (i, 0)),
        ],
        out_shape=[
            jax.ShapeDtypeStruct((_NP, _HF), jnp.float32),
            jax.ShapeDtypeStruct((_NP, _HF), jnp.float32),
        ],
    )(s0, s1, gl, gh, dinv, b, w)


def _pool_body(s0_ref, s1_ref, gl_ref, gh_ref, dinv_ref, b_ref, batch_ref,
               wp_ref, bp_ref, sums_ref, cnt_ref, out_ref, *, nblk):
    i = pl.program_id(0)

    @pl.when(i == 0)
    def _init():
        sums_ref[...] = jnp.zeros_like(sums_ref)
        cnt_ref[...] = jnp.zeros_like(cnt_ref)

    h = jnp.concatenate(
        [s0_ref[...] + gl_ref[...], s1_ref[...] + gh_ref[...]], axis=1)
    h = h * dinv_ref[...] + b_ref[...]
    v = jnp.dot(h, wp_ref[...],
                preferred_element_type=jnp.float32, precision=_HIGH)
    ids = lax.broadcasted_iota(jnp.int32, (1, _G), 1)
    oh = (batch_ref[...] == ids).astype(jnp.float32)
    sums_ref[...] += lax.dot_general(
        oh, v, (((0,), (0,)), ((), ())),
        preferred_element_type=jnp.float32, precision=_HIGH)
    cnt_ref[...] += jnp.sum(oh, axis=0)[:, None]

    @pl.when(i == nblk - 1)
    def _epilogue():
        out_ref[...] = (sums_ref[...] / jnp.maximum(cnt_ref[...], 1.0)
                        + bp_ref[...])


def _pool(s0, s1, gl, gh, dinv, b, batch_pad, wp, bp):
    nblk = _NP // _BR
    body = functools.partial(_pool_body, nblk=nblk)
    _, _, out = pl.pallas_call(
        body,
        grid=(nblk,),
        in_specs=[
            pl.BlockSpec((_BR, _HF), lambda i: (i, 0)),
            pl.BlockSpec((_BR, _HF), lambda i: (i, 0)),
            pl.BlockSpec((_BR, _HF), lambda i: (i, 0)),
            pl.BlockSpec((_BR, _HF), lambda i: (i, 0)),
            pl.BlockSpec((_BR, 1), lambda i: (i, 0)),
            pl.BlockSpec((1, _DH), lambda i: (0, 0)),
            pl.BlockSpec((_BR, 1), lambda i: (i, 0)),
            pl.BlockSpec((_DH, 1), lambda i: (0, 0)),
            pl.BlockSpec((1, 1), lambda i: (0, 0)),
        ],
        out_specs=[
            pl.BlockSpec((_G, 1), lambda i: (0, 0)),
            pl.BlockSpec((_G, 1), lambda i: (0, 0)),
            pl.BlockSpec((_G, 1), lambda i: (0, 0)),
        ],
        out_shape=[
            jax.ShapeDtypeStruct((_G, 1), jnp.float32),
            jax.ShapeDtypeStruct((_G, 1), jnp.float32),
            jax.ShapeDtypeStruct((_G, 1), jnp.float32),
        ],
    )(s0, s1, gl, gh, dinv, b, batch_pad, wp, bp)
    return out


def kernel(x, edge_index, batch, W1, b1, W2, b2, W3, b3, Wp, bp):
    src, dst = edge_index[0], edge_index[1]

    deg0, deg1 = _sc_deg(dst)

    x_pad = jnp.zeros((_NP, 128), jnp.float32).at[:_N].set(x)
    dinv, gl, gh = _prep1(x_pad, W1, deg0[:, :1], deg1[:, :1])

    s0, s1 = _sc_agg(gl, gh, src, dst)
    gl, gh = _fuse(s0, s1, gl, gh, dinv, b1[None, :], W2)
    s0, s1 = _sc_agg(gl, gh, src, dst)
    gl, gh = _fuse(s0, s1, gl, gh, dinv, b2[None, :], W3)
    s0, s1 = _sc_agg(gl, gh, src, dst)

    batch_pad = jnp.full((_NP, 1), _G, jnp.int32).at[:_N, 0].set(batch)
    return _pool(s0, s1, gl, gh, dinv, b3[None, :], batch_pad, Wp,
                 bp[:, None])
